# baseline (speedup 1.0000x reference)
import functools

import jax
import jax.numpy as jnp
from jax import lax
from jax.experimental import pallas as pl
from jax.experimental.pallas import tpu as pltpu
from jax.experimental.pallas import tpu_sc as plsc


def kernel(input_ids, embedding_table, pos_emb_table):
    B, L = input_ids.shape
    V, D = embedding_table.shape
    N = B * L
    NW = 32
    n_per_w = N // NW
    CHUNK = L
    n_chunks = n_per_w // CHUNK

    ids_flat = input_ids.reshape(N).astype(jnp.int32)
    mesh = plsc.VectorSubcoreMesh(core_axis_name="c", subcore_axis_name="s")

    @functools.partial(
        pl.kernel,
        mesh=mesh,
        out_type=jax.ShapeDtypeStruct((N, D), jnp.float32),
        scratch_types=[
            pltpu.VMEM((n_per_w,), jnp.int32),
            pltpu.VMEM((L, D), jnp.float32),
            pltpu.VMEM((CHUNK, D), jnp.float32),
            pltpu.SemaphoreType.DMA,
        ],
    )
    def emb_kernel(ids_hbm, tab_hbm, pos_hbm, out_hbm,
                   idx_v, pos_v, rowbuf, sem):
        wid = lax.axis_index("s") * 2 + lax.axis_index("c")
        base = wid * n_per_w
        pltpu.sync_copy(ids_hbm.at[pl.ds(base, n_per_w)], idx_v)
        pltpu.sync_copy(pos_hbm, pos_v)

        def add_row(r, carry):
            for j in range(D // 16):
                s = pl.ds(j * 16, 16)
                rowbuf[r, s] = rowbuf[r, s] + pos_v[r, s]
            return carry

        def body(c, carry):
            off = pl.multiple_of(c * CHUNK, CHUNK)
            pltpu.async_copy(tab_hbm.at[idx_v.at[pl.ds(off, CHUNK)]],
                             rowbuf, sem).wait()
            lax.fori_loop(0, CHUNK, add_row, 0)
            pltpu.sync_copy(rowbuf, out_hbm.at[pl.ds(base + off, CHUNK)])
            return carry

        lax.fori_loop(0, n_chunks, body, 0)

    out = emb_kernel(ids_flat, embedding_table, pos_emb_table)
    return out.reshape(B, L, D)
